# baseline (speedup 1.0000x reference)
import jax
import jax.numpy as jnp
import numpy as np
from jax import lax
from jax.experimental import pallas as pl
from jax.experimental.pallas import tpu as pltpu
from jax.experimental.pallas import tpu_sc as plsc

NC = 2
NS = 16
NW = NC * NS
L = 16

def _lane_consts():
    lane = lax.iota(jnp.int32, L)
    conds = {k: (lane & k) == 0 for k in (8, 4, 2, 1)}
    perms = {k: lane ^ k for k in (8, 4, 2, 1)}
    bitrev = ((lane & 1) << 3) | ((lane & 2) << 1) | ((lane & 4) >> 1) | (
        (lane & 8) >> 3)
    return conds, perms, bitrev


def _hsum16(vecs, conds, perms, bitrev):
    for k in (8, 4, 2, 1):
        cond, perm = conds[k], perms[k]
        nxt = []
        for i in range(0, len(vecs), 2):
            x, y = vecs[i], vecs[i + 1]
            a = jnp.where(cond, x, y)
            c = jnp.where(cond, y, x)
            nxt.append(a + jnp.take_along_axis(c, perm, axis=0))
        vecs = nxt
    return jnp.take_along_axis(vecs[0], bitrev, axis=0)


def _gmf_kernel(uidx_hbm, iidx_hbm, utab_hbm, itab_hbm, w_hbm, b_hbm,
                out_hbm, uidx_v, iidx_v, urows_v, irows_v, res_v, w_v, b_v,
                sem):
    bpw = res_v.shape[0]
    wid = lax.axis_index("s") * NC + lax.axis_index("c")
    base = wid * bpw

    pltpu.sync_copy(w_hbm, w_v)
    pltpu.sync_copy(b_hbm, b_v)
    pltpu.sync_copy(uidx_hbm.at[pl.ds(base, bpw)], uidx_v)
    pltpu.sync_copy(iidx_hbm.at[pl.ds(base, bpw)], iidx_v)
    cu = pltpu.async_copy(utab_hbm.at[uidx_v], urows_v, sem)
    ci = pltpu.async_copy(itab_hbm.at[iidx_v], irows_v, sem)
    cu.wait()
    ci.wait()

    w_lo = w_v[pl.ds(0, L)]
    w_hi = w_v[pl.ds(L, L)]
    bvec = b_v[...]
    conds, perms, bitrev = _lane_consts()

    def group(r, carry):
        off = r * L
        ps = []
        for j in range(L):
            row = off + j
            u_lo = urows_v[row, pl.ds(0, L)]
            u_hi = urows_v[row, pl.ds(L, L)]
            i_lo = irows_v[row, pl.ds(0, L)]
            i_hi = irows_v[row, pl.ds(L, L)]
            ps.append(u_lo * i_lo * w_lo + u_hi * i_hi * w_hi)
        z = _hsum16(ps, conds, perms, bitrev) + bvec
        res_v[pl.ds(off, L)] = 1.0 / (1.0 + jnp.exp(-z))
        return carry

    lax.fori_loop(0, bpw // L, group, 0)
    pltpu.sync_copy(res_v, out_hbm.at[pl.ds(base, bpw)])


def kernel(user_input, item_input, user_table, item_table, W, b):
    B = user_input.shape[0]
    D = user_table.shape[1]
    assert D == 2 * L and B % (NW * L) == 0
    bpw = B // NW

    uidx = user_input.astype(jnp.int32)
    iidx = item_input.astype(jnp.int32)
    w_flat = W.reshape(D).astype(jnp.float32)
    bvec = jnp.broadcast_to(b.astype(jnp.float32).reshape(1), (L,))

    mesh = plsc.VectorSubcoreMesh(core_axis_name="c", subcore_axis_name="s")
    run = pl.kernel(
        _gmf_kernel,
        out_type=jax.ShapeDtypeStruct((B,), jnp.float32),
        mesh=mesh,
        scratch_types=[
            pltpu.VMEM((bpw,), jnp.int32),
            pltpu.VMEM((bpw,), jnp.int32),
            pltpu.VMEM((bpw, D), jnp.float32),
            pltpu.VMEM((bpw, D), jnp.float32),
            pltpu.VMEM((bpw,), jnp.float32),
            pltpu.VMEM((D,), jnp.float32),
            pltpu.VMEM((L,), jnp.float32),
            pltpu.SemaphoreType.DMA,
        ],
        compiler_params=pltpu.CompilerParams(use_tc_tiling_on_sc=False),
    )
    out = run(uidx, iidx, user_table, item_table, w_flat, bvec)
    return out.reshape(B, 1)
